# baseline (speedup 1.0000x reference)
import dataclasses
import functools

import jax
import jax.numpy as jnp
from jax import lax
from jax.experimental import pallas as pl
from jax.experimental.pallas import tpu as pltpu
from jax.experimental.pallas import tpu_sc as plsc

N_FACES = 100000
N_EDGES = 150000
NNZ = 300000
D = 128

NSUB = 16
NCORE = 2
LANES = 16
K = 128
CHUNK = 8192
ROWS_PER_TILE = CHUNK // NSUB
NCH1 = -(-N_EDGES // CHUNK)
NCH2 = -(-N_FACES // CHUNK)
P1 = NCH1 * CHUNK
P2 = NCH2 * CHUNK
NQ = 2
SB = 8
BIGDEST = 1 << 29

f32 = jnp.float32
i32 = jnp.int32


def _mm_pm_body(sigmoid_in, x_ref, w_ref, o_ref):
    xb = x_ref[...]
    if sigmoid_in:
        xb = jax.nn.sigmoid(xb)
    r = jnp.dot(xb, w_ref[...], preferred_element_type=f32)
    o_ref[0] = r
    o_ref[1] = -r


def _mm_pm(x, w, bm, sigmoid_in):
    n = x.shape[0]
    nblk = n // bm
    out = pl.pallas_call(
        functools.partial(_mm_pm_body, sigmoid_in),
        grid=(nblk,),
        in_specs=[
            pl.BlockSpec((bm, D), lambda i: (i, 0)),
            pl.BlockSpec((D, D), lambda i: (0, 0)),
        ],
        out_specs=pl.BlockSpec((2, bm, D), lambda i: (0, i, 0)),
        out_shape=jax.ShapeDtypeStruct((2, n, D), f32),
    )(x, w)
    return out.reshape(2 * n, D)


def _sigmoid_body(x_ref, o_ref):
    o_ref[...] = jax.nn.sigmoid(x_ref[...])


def _sigmoid_head(x, n, bm):
    return pl.pallas_call(
        _sigmoid_body,
        grid=(n // bm,),
        in_specs=[pl.BlockSpec((bm, D), lambda i: (i, 0))],
        out_specs=pl.BlockSpec((bm, D), lambda i: (i, 0)),
        out_shape=jax.ShapeDtypeStruct((n, D), f32),
    )(x)


def _vext(vec, c):
    sel = jnp.where(lax.iota(i32, LANES) == c, vec, -(1 << 30))
    return jnp.max(sel)


def _make_spmm(nch, out_rows):
    mesh = plsc.VectorSubcoreMesh(core_axis_name="c", subcore_axis_name="s")
    nloop = -(-nch // NCORE)
    cp = pltpu.CompilerParams()
    if "needs_layout_passes" in pltpu.CompilerParams.__dataclass_fields__:
        cp = dataclasses.replace(cp, needs_layout_passes=False)

    @functools.partial(
        pl.kernel,
        mesh=mesh,
        compiler_params=cp,
        out_type=jax.ShapeDtypeStruct((out_rows, D), f32),
        scratch_types=[
            pltpu.VMEM((32,), i32),
            pltpu.VMEM((SB * K,), i32),
            pltpu.VMEM((SB * K,), i32),
        ] + [pltpu.VMEM((K,), i32)] * SB + [
            pltpu.VMEM((K, D), f32),
            pltpu.VMEM((K, D), f32),
            pltpu.VMEM((K, D), f32),
            pltpu.VMEM_SHARED((CHUNK + 8, D), f32),
            pltpu.SemaphoreType.DMA,
            pltpu.SemaphoreType.DMA,
            pltpu.SemaphoreType.DMA,
        ],
    )
    def spmm(src_hbm, gidx_hbm, didx_hbm, bounds_hbm, zeros_hbm, out_hbm,
             bnd_v, gi_v, di_v, r0, r1, r2, r3, r4, r5, r6, r7,
             gat_0, gat_1, zero_v, acc, sem_g, sem_s0, sem_s1):
        rel_refs = (r0, r1, r2, r3, r4, r5, r6, r7)
        gat_refs = (gat_0, gat_1)
        sc_sems = (sem_s0, sem_s1)
        core = lax.axis_index("c")
        sub = lax.axis_index("s")

        pltpu.sync_copy(bounds_hbm, bnd_v)
        pltpu.sync_copy(zeros_hbm, zero_v)
        b0 = bnd_v[pl.ds(0, LANES)]
        b1 = bnd_v[pl.ds(LANES, LANES)]

        def bound(c):
            return jnp.where(c < LANES, _vext(b0, c), _vext(b1, c - LANES))

        def chunk_body(it, carry):
            c = core + NCORE * it

            @pl.when(c < nch)
            def _():
                base = pl.multiple_of(c * CHUNK, CHUNK)
                for q in range(ROWS_PER_TILE // K):
                    pltpu.sync_copy(
                        zero_v,
                        acc.at[pl.ds(pl.multiple_of(
                            sub * ROWS_PER_TILE + q * K, K), K)])
                plsc.subcore_barrier()

                lo_c = pl.multiple_of(bound(c) & ~7, 8)
                hi_c = pl.multiple_of((bound(c + 1) + 7) & ~7, 8)
                ln = hi_c - lo_c
                lo = pl.multiple_of(lo_c + ((ln * sub) // NSUB & ~7), 8)
                hi = pl.multiple_of(lo_c + ((ln * (sub + 1)) // NSUB & ~7), 8)
                nsb = (hi - lo + SB * K - 1) // (SB * K)

                def sblock(t, carry2):
                    pos = pl.multiple_of(lo + t * (SB * K), 8)
                    pltpu.sync_copy(gidx_hbm.at[pl.ds(pos, SB * K)], gi_v)
                    pltpu.sync_copy(didx_hbm.at[pl.ds(pos, SB * K)], di_v)
                    for k in range(SB):
                        for kk in range(K // LANES):
                            o = k * K + kk * LANES
                            d = di_v[pl.ds(o, LANES)]
                            gpos = pos + o + lax.iota(i32, LANES)
                            rel = d - (base + 1)
                            ok = (rel >= 0) & (rel < CHUNK) & (gpos < hi)
                            rel_refs[k][pl.ds(kk * LANES, LANES)] = (
                                jnp.where(ok, rel, CHUNK))
                    gh = [None] * SB
                    sh = [None] * SB
                    for k in range(SB):
                        if k >= NQ:
                            sh[k - NQ].wait()
                        gh[k] = pltpu.async_copy(
                            src_hbm.at[gi_v.at[pl.ds(k * K, K)]],
                            gat_refs[k % NQ], sem_g)
                        if k >= 1:
                            gh[k - 1].wait()
                            sh[k - 1] = pltpu.async_copy(
                                gat_refs[(k - 1) % NQ],
                                acc.at[rel_refs[k - 1]],
                                sc_sems[(k - 1) % NQ], add=True)
                    gh[SB - 1].wait()
                    sh[SB - 1] = pltpu.async_copy(
                        gat_refs[(SB - 1) % NQ], acc.at[rel_refs[SB - 1]],
                        sc_sems[(SB - 1) % NQ], add=True)
                    sh[SB - 2].wait()
                    sh[SB - 1].wait()
                    return carry2

                lax.fori_loop(0, nsb, sblock, 0)
                plsc.subcore_barrier()

                pltpu.sync_copy(
                    acc.at[pl.ds(pl.multiple_of(sub * ROWS_PER_TILE,
                                                ROWS_PER_TILE), ROWS_PER_TILE)],
                    out_hbm.at[pl.ds(pl.multiple_of(
                        base + sub * ROWS_PER_TILE, ROWS_PER_TILE),
                        ROWS_PER_TILE)])
            return carry

        lax.fori_loop(0, nloop, chunk_body, 0)

    return spmm


@functools.cache
def _get_spmm(nch, out_rows):
    return _make_spmm(nch, out_rows)


RB = 256
RSUB = 8
SHIFT = 13
NBKT = 32
NNZ2 = 2 * NNZ
RBLK = -(-NNZ2 // (RB * RSUB))
NNZR = RBLK * RB * RSUB
NNZP2 = NNZ2 + 1088


def _rank_body(d_ref, rank_ref, tot_ref, run_ref):
    step = pl.program_id(0)

    @pl.when(step == 0)
    def _():
        run_ref[...] = jnp.zeros((1, NBKT), f32)

    r2d = lax.broadcasted_iota(i32, (RB, RB), 0)
    c2d = lax.broadcasted_iota(i32, (RB, RB), 1)
    sut = (r2d < c2d).astype(jnp.bfloat16)
    kblk = d_ref[...]
    b = jnp.minimum(lax.shift_right_logical(kblk, SHIFT), NBKT - 1)
    c3 = lax.broadcasted_iota(i32, (RSUB, NBKT, RB), 1)
    y = (b[:, None, :] == c3)
    yf = y.astype(f32)
    xt = y.reshape(RB, RB).astype(jnp.bfloat16)
    csum_t = jnp.dot(xt, sut, preferred_element_type=f32)
    tj = jnp.sum(yf, axis=2)
    acc = run_ref[...]
    goffs_rows = []
    for j in range(RSUB):
        goffs_rows.append(acc)
        acc = acc + tj[j:j + 1, :]
    goffs = jnp.concatenate(goffs_rows, axis=0)
    csum3 = csum_t.reshape(RSUB, NBKT, RB)
    rank = jnp.sum((csum3 + goffs[:, :, None]) * yf, axis=1)
    rank_ref[...] = rank.astype(i32)
    run_ref[...] = acc

    @pl.when(step == pl.num_programs(0) - 1)
    def _():
        tot_ref[...] = run_ref[...].astype(i32)


def _rank(keys):
    kpad = jnp.concatenate(
        [keys, jnp.full((NNZR - NNZ2,), BIGDEST, i32)])
    kpad = kpad.reshape(RBLK * RSUB, RB)
    rank, tot = pl.pallas_call(
        _rank_body,
        grid=(RBLK,),
        in_specs=[pl.BlockSpec((RSUB, RB), lambda i: (i, 0))],
        out_specs=[pl.BlockSpec((RSUB, RB), lambda i: (i, 0)),
                   pl.BlockSpec((1, NBKT), lambda i: (0, 0))],
        out_shape=[jax.ShapeDtypeStruct((RBLK * RSUB, RB), i32),
                   jax.ShapeDtypeStruct((1, NBKT), i32)],
        scratch_shapes=[pltpu.VMEM((1, NBKT), f32)],
    )(kpad)
    return rank.reshape(NNZR)[:NNZ2], tot[0]


def _bucket2(rows, cols, vals):
    rows = rows.astype(i32)
    cols = cols.astype(i32)
    keys = jnp.concatenate([rows, cols + 19 * CHUNK])
    rank, tot = _rank(keys)
    offs = jnp.concatenate([jnp.zeros((1,), i32),
                            jnp.cumsum(tot, dtype=i32)[:-1]])
    b = jnp.minimum(lax.shift_right_logical(keys, SHIFT), NBKT - 1)
    pos = offs[b] + rank
    neg = (vals < 0).astype(i32)
    gall = jnp.concatenate([cols + N_FACES * neg, rows + P1 * neg])
    dall = jnp.concatenate([rows, cols])
    pairs = jnp.stack([dall + 1, gall], axis=1)
    dg = jnp.zeros((NNZP2, 2), i32).at[pos].add(pairs, unique_indices=True)
    d = dg[:, 0]
    g = dg[:, 1]
    bounds1 = jnp.concatenate(
        [offs[:NCH1 + 1], jnp.full((32 - NCH1 - 1,), NNZ, i32)])
    bounds2 = jnp.concatenate(
        [offs[19:], jnp.full((32 - 13,), NNZR, i32)])
    return g, d, bounds1, bounds2


def kernel(x, rows, cols, vals, W1, W2):
    gidx, didx, bounds1, bounds2 = _bucket2(rows, cols, vals)
    zeros = jnp.zeros((K, D), f32)

    hpm = _mm_pm(x, W1, 1000, sigmoid_in=False)
    he = _get_spmm(NCH1, P1)(hpm, gidx, didx, bounds1, zeros)
    h2pm = _mm_pm(he, W2, 512, sigmoid_in=True)
    out = _get_spmm(NCH2, P2)(h2pm, gidx, didx, bounds2, zeros)
    return _sigmoid_head(out, N_FACES, 1000)


# speedup vs baseline: 1.9562x; 1.9562x over previous
import dataclasses
import functools

import jax
import jax.numpy as jnp
from jax import lax
from jax.experimental import pallas as pl
from jax.experimental.pallas import tpu as pltpu
from jax.experimental.pallas import tpu_sc as plsc

N_FACES = 100000
N_EDGES = 150000
NNZ = 300000
D = 128

NSUB = 16
NCORE = 2
LANES = 16
K = 128
CHUNK = 8192
ROWS_PER_TILE = CHUNK // NSUB
NCH1 = -(-N_EDGES // CHUNK)
NCH2 = -(-N_FACES // CHUNK)
P1 = NCH1 * CHUNK
P2 = NCH2 * CHUNK
NQ = 2
SB = 8
BIGDEST = 1 << 29

f32 = jnp.float32
i32 = jnp.int32


def _mm_pm_body(sigmoid_in, x_ref, w_ref, o_ref):
    xb = x_ref[...]
    if sigmoid_in:
        xb = jax.nn.sigmoid(xb)
    r = jnp.dot(xb, w_ref[...], preferred_element_type=f32)
    o_ref[0] = r
    o_ref[1] = -r


def _mm_pm(x, w, bm, sigmoid_in):
    n = x.shape[0]
    nblk = n // bm
    out = pl.pallas_call(
        functools.partial(_mm_pm_body, sigmoid_in),
        grid=(nblk,),
        in_specs=[
            pl.BlockSpec((bm, D), lambda i: (i, 0)),
            pl.BlockSpec((D, D), lambda i: (0, 0)),
        ],
        out_specs=pl.BlockSpec((2, bm, D), lambda i: (0, i, 0)),
        out_shape=jax.ShapeDtypeStruct((2, n, D), f32),
    )(x, w)
    return out.reshape(2 * n, D)


def _sigmoid_body(x_ref, o_ref):
    o_ref[...] = jax.nn.sigmoid(x_ref[...])


def _sigmoid_head(x, n, bm):
    return pl.pallas_call(
        _sigmoid_body,
        grid=(n // bm,),
        in_specs=[pl.BlockSpec((bm, D), lambda i: (i, 0))],
        out_specs=pl.BlockSpec((bm, D), lambda i: (i, 0)),
        out_shape=jax.ShapeDtypeStruct((n, D), f32),
    )(x)


def _vext(vec, c):
    sel = jnp.where(lax.iota(i32, LANES) == c, vec, -(1 << 30))
    return jnp.max(sel)


def _make_spmm(nch, out_rows):
    mesh = plsc.VectorSubcoreMesh(core_axis_name="c", subcore_axis_name="s")
    nloop = -(-nch // NCORE)
    cp = pltpu.CompilerParams()
    if "needs_layout_passes" in pltpu.CompilerParams.__dataclass_fields__:
        cp = dataclasses.replace(cp, needs_layout_passes=False)

    @functools.partial(
        pl.kernel,
        mesh=mesh,
        compiler_params=cp,
        out_type=jax.ShapeDtypeStruct((out_rows, D), f32),
        scratch_types=[
            pltpu.VMEM((32,), i32),
            pltpu.VMEM((NQ * K,), i32),
            pltpu.VMEM((NQ * K,), i32),
        ] + [pltpu.VMEM((K,), i32)] * NQ + [
            pltpu.VMEM((K, D), f32),
            pltpu.VMEM((K, D), f32),
            pltpu.VMEM((K, D), f32),
            pltpu.VMEM_SHARED((CHUNK + 8, D), f32),
            pltpu.SemaphoreType.DMA,
            pltpu.SemaphoreType.DMA,
            pltpu.SemaphoreType.DMA,
        ],
    )
    def spmm(src_hbm, gidx_hbm, didx_hbm, bounds_hbm, zeros_hbm, out_hbm,
             bnd_v, gi_v, di_v, r0, r1,
             gat_0, gat_1, zero_v, acc, sem_g, sem_s0, sem_s1):
        rel_refs = (r0, r1)
        gat_refs = (gat_0, gat_1)
        sc_sems = (sem_s0, sem_s1)
        core = lax.axis_index("c")
        sub = lax.axis_index("s")

        pltpu.sync_copy(bounds_hbm, bnd_v)
        pltpu.sync_copy(zeros_hbm, zero_v)
        b0 = bnd_v[pl.ds(0, LANES)]
        b1 = bnd_v[pl.ds(LANES, LANES)]

        def bound(c):
            return jnp.where(c < LANES, _vext(b0, c), _vext(b1, c - LANES))

        def chunk_body(it, carry):
            c = core + NCORE * it

            @pl.when(c < nch)
            def _():
                base = pl.multiple_of(c * CHUNK, CHUNK)
                for q in range(ROWS_PER_TILE // K):
                    pltpu.sync_copy(
                        zero_v,
                        acc.at[pl.ds(pl.multiple_of(
                            sub * ROWS_PER_TILE + q * K, K), K)])
                plsc.subcore_barrier()

                lo_c = pl.multiple_of(bound(c) & ~7, 8)
                hi_c = pl.multiple_of((bound(c + 1) + 7) & ~7, 8)
                ln = hi_c - lo_c
                lo = pl.multiple_of(lo_c + ((ln * sub) // NSUB & ~7), 8)
                hi = pl.multiple_of(lo_c + ((ln * (sub + 1)) // NSUB & ~7), 8)
                nb = (hi - lo + NQ * K - 1) // (NQ * K)

                def batch(t, carry2):
                    pos = pl.multiple_of(lo + t * (NQ * K), 8)
                    pltpu.sync_copy(gidx_hbm.at[pl.ds(pos, NQ * K)], gi_v)
                    pltpu.sync_copy(didx_hbm.at[pl.ds(pos, NQ * K)], di_v)
                    gh = [pltpu.async_copy(
                        src_hbm.at[gi_v.at[pl.ds(q * K, K)]],
                        gat_refs[q], sem_g) for q in range(NQ)]
                    for q in range(NQ):
                        for kk in range(K // LANES):
                            o = q * K + kk * LANES
                            d = di_v[pl.ds(o, LANES)]
                            gpos = pos + o + lax.iota(i32, LANES)
                            rel = d - (base + 1)
                            ok = (rel >= 0) & (rel < CHUNK) & (gpos < hi)
                            rel_refs[q][pl.ds(kk * LANES, LANES)] = (
                                jnp.where(ok, rel, CHUNK))
                    sh = []
                    for q in range(NQ):
                        gh[q].wait()
                        sh.append(pltpu.async_copy(
                            gat_refs[q], acc.at[rel_refs[q]],
                            sc_sems[q], add=True))
                    for h in sh:
                        h.wait()
                    return carry2

                lax.fori_loop(0, nb, batch, 0)
                plsc.subcore_barrier()

                pltpu.sync_copy(
                    acc.at[pl.ds(pl.multiple_of(sub * ROWS_PER_TILE,
                                                ROWS_PER_TILE), ROWS_PER_TILE)],
                    out_hbm.at[pl.ds(pl.multiple_of(
                        base + sub * ROWS_PER_TILE, ROWS_PER_TILE),
                        ROWS_PER_TILE)])
            return carry

        lax.fori_loop(0, nloop, chunk_body, 0)

    return spmm


@functools.cache
def _get_spmm(nch, out_rows):
    return _make_spmm(nch, out_rows)


RB = 256
RSUB = 8
SHIFT = 13
NBKT = 32
NNZ2 = 2 * NNZ
RBLK = -(-NNZ2 // (RB * RSUB))
NNZR = RBLK * RB * RSUB
NNZP2 = NNZ2 + 1088


def _rank_body(d_ref, rank_ref, tot_ref, run_ref):
    step = pl.program_id(0)

    @pl.when(step == 0)
    def _():
        run_ref[...] = jnp.zeros((1, NBKT), f32)

    r2d = lax.broadcasted_iota(i32, (RB, RB), 0)
    c2d = lax.broadcasted_iota(i32, (RB, RB), 1)
    sut = (r2d < c2d).astype(jnp.bfloat16)
    kblk = d_ref[...]
    b = jnp.minimum(lax.shift_right_logical(kblk, SHIFT), NBKT - 1)
    c3 = lax.broadcasted_iota(i32, (RSUB, NBKT, RB), 1)
    y = (b[:, None, :] == c3)
    yf = y.astype(f32)
    xt = y.reshape(RB, RB).astype(jnp.bfloat16)
    csum_t = jnp.dot(xt, sut, preferred_element_type=f32)
    tj = jnp.sum(yf, axis=2)
    acc = run_ref[...]
    goffs_rows = []
    for j in range(RSUB):
        goffs_rows.append(acc)
        acc = acc + tj[j:j + 1, :]
    goffs = jnp.concatenate(goffs_rows, axis=0)
    csum3 = csum_t.reshape(RSUB, NBKT, RB)
    rank = jnp.sum((csum3 + goffs[:, :, None]) * yf, axis=1)
    rank_ref[...] = rank.astype(i32)
    run_ref[...] = acc

    @pl.when(step == pl.num_programs(0) - 1)
    def _():
        tot_ref[...] = run_ref[...].astype(i32)


def _rank(keys):
    kpad = jnp.concatenate(
        [keys, jnp.full((NNZR - NNZ2,), BIGDEST, i32)])
    kpad = kpad.reshape(RBLK * RSUB, RB)
    rank, tot = pl.pallas_call(
        _rank_body,
        grid=(RBLK,),
        in_specs=[pl.BlockSpec((RSUB, RB), lambda i: (i, 0))],
        out_specs=[pl.BlockSpec((RSUB, RB), lambda i: (i, 0)),
                   pl.BlockSpec((1, NBKT), lambda i: (0, 0))],
        out_shape=[jax.ShapeDtypeStruct((RBLK * RSUB, RB), i32),
                   jax.ShapeDtypeStruct((1, NBKT), i32)],
        scratch_shapes=[pltpu.VMEM((1, NBKT), f32)],
    )(kpad)
    return rank.reshape(NNZR)[:NNZ2], tot[0]


def _bucket2(rows, cols, vals):
    rows = rows.astype(i32)
    cols = cols.astype(i32)
    keys = jnp.concatenate([rows, cols + 19 * CHUNK])
    rank, tot = _rank(keys)
    offs = jnp.concatenate([jnp.zeros((1,), i32),
                            jnp.cumsum(tot, dtype=i32)[:-1]])
    b = jnp.minimum(lax.shift_right_logical(keys, SHIFT), NBKT - 1)
    pos = offs[b] + rank
    neg = (vals < 0).astype(i32)
    gall = jnp.concatenate([cols + N_FACES * neg, rows + P1 * neg])
    dall = jnp.concatenate([rows, cols])
    d = jnp.zeros((NNZP2,), i32).at[pos].add(dall + 1, unique_indices=True)
    g = jnp.zeros((NNZP2,), i32).at[pos].add(gall, unique_indices=True)
    bounds1 = jnp.concatenate(
        [offs[:NCH1 + 1], jnp.full((32 - NCH1 - 1,), NNZ, i32)])
    bounds2 = jnp.concatenate(
        [offs[19:], jnp.full((32 - 13,), NNZR, i32)])
    return g, d, bounds1, bounds2


def kernel(x, rows, cols, vals, W1, W2):
    gidx, didx, bounds1, bounds2 = _bucket2(rows, cols, vals)
    zeros = jnp.zeros((K, D), f32)

    hpm = _mm_pm(x, W1, 1000, sigmoid_in=False)
    he = _get_spmm(NCH1, P1)(hpm, gidx, didx, bounds1, zeros)
    h2pm = _mm_pm(he, W2, 512, sigmoid_in=True)
    out = _get_spmm(NCH2, P2)(h2pm, gidx, didx, bounds2, zeros)
    return _sigmoid_head(out, N_FACES, 1000)
